# Initial kernel scaffold; baseline (speedup 1.0000x reference)
import jax
import jax.numpy as jnp
from jax.experimental import pallas as pl


def kernel(x, w1, b1, s1, w2, b2, s2, wfc, fcb):
    raise NotImplementedError("write your pallas kernel here")


# speedup vs baseline: 2.8564x; 2.8564x over previous
import jax
import jax.numpy as jnp
from jax.experimental import pallas as pl
from jax.experimental.pallas import tpu as pltpu

BLK = 64
H1 = 32
W1 = 30
H2 = 16
NL = 128


def _body(x_ref, w1_ref, b1_ref, w2_ref, b2_ref, wfc_ref, fcb_ref, o_ref,
          s_ref, a1_ref, s3_ref, f_ref):
    m0 = BLK * H1 - 2
    m1 = BLK * H2 - 2
    r1 = BLK * H2
    r2 = BLK * 8

    h1 = jnp.dot(x_ref[pl.ds(0, m0), :], w1_ref[0],
                 preferred_element_type=jnp.float32)
    for ky in (1, 2):
        h1 = h1 + jnp.dot(x_ref[pl.ds(ky, m0), :], w1_ref[ky],
                          preferred_element_type=jnp.float32)
    s_ref[pl.ds(0, m0), :] = jnp.maximum(h1[:, :NL], h1[:, NL:]) + b1_ref[...]

    ev = s_ref[pl.ds(0, r1, 2), :]
    od = s_ref[pl.ds(1, r1, 2), :]
    row16 = jax.lax.broadcasted_iota(jnp.int32, (r1, NL), 0) & (H2 - 1)
    a1 = jnp.where(row16 < 14, jnp.maximum(jnp.maximum(ev, od), 0.0), 0.0)
    a1_ref[pl.ds(1, r1), :] = a1
    a1_ref[pl.ds(0, 1), :] = jnp.zeros((1, NL), jnp.float32)

    a1c = jnp.concatenate(
        [a1_ref[pl.ds(ky, m1), :] for ky in (0, 1, 2)], axis=1)
    h2 = jnp.dot(a1c, w2_ref[...], preferred_element_type=jnp.float32)
    s3_ref[pl.ds(0, m1), :] = jnp.maximum(h2[:, :NL], h2[:, NL:]) + b2_ref[...]

    ev2 = s3_ref[pl.ds(0, r2, 2), :]
    od2 = s3_ref[pl.ds(1, r2, 2), :]
    row8 = jax.lax.broadcasted_iota(jnp.int32, (r2, NL), 0) & 7
    f_ref[...] = jnp.where(row8 < 7,
                           jnp.maximum(jnp.maximum(ev2, od2), 0.0), 0.0)

    fc = jnp.concatenate(
        [f_ref[pl.ds(y2, BLK, 8), :] for y2 in range(7)], axis=1)
    o_ref[...] = jnp.dot(fc, wfc_ref[...],
                         preferred_element_type=jnp.float32) + fcb_ref[...]


def kernel(x, w1, b1, s1, w2, b2, s2, wfc, fcb):
    del s1, s2
    n = x.shape[0]
    n_pad = ((n + BLK - 1) // BLK) * BLK
    xp = jnp.pad(x[:, 0], ((0, n_pad - n), (1, H1 - 29), (1, 1)))
    x2d = xp.reshape(n_pad * H1, W1)

    w2s = jnp.concatenate([w2[0], w2[1], w2[2]], axis=0)
    wfcs = jnp.concatenate([wfc[i] for i in range(7)], axis=0)

    logits = pl.pallas_call(
        _body,
        out_shape=jax.ShapeDtypeStruct((n_pad, NL), jnp.float32),
        grid=(n_pad // BLK,),
        in_specs=[
            pl.BlockSpec((BLK * H1, W1), lambda i: (i, 0)),
            pl.BlockSpec((3, W1, 2 * NL), lambda i: (0, 0, 0)),
            pl.BlockSpec((1, NL), lambda i: (0, 0)),
            pl.BlockSpec((3 * NL, 2 * NL), lambda i: (0, 0)),
            pl.BlockSpec((1, NL), lambda i: (0, 0)),
            pl.BlockSpec((7 * NL, NL), lambda i: (0, 0)),
            pl.BlockSpec((1, NL), lambda i: (0, 0)),
        ],
        out_specs=pl.BlockSpec((BLK, NL), lambda i: (i, 0)),
        scratch_shapes=[
            pltpu.VMEM((BLK * H1, NL), jnp.float32),
            pltpu.VMEM((BLK * H2 + 8, NL), jnp.float32),
            pltpu.VMEM((BLK * H2, NL), jnp.float32),
            pltpu.VMEM((BLK * 8, NL), jnp.float32),
        ],
        compiler_params=pltpu.CompilerParams(
            dimension_semantics=("parallel",)),
    )(x2d, w1, b1, w2s, b2, wfcs, fcb)
    return logits[:n, :10]
